# baseline (speedup 1.0000x reference)
import functools

import jax
import jax.numpy as jnp
from jax import lax
from jax.experimental import pallas as pl
from jax.experimental.pallas import tpu as pltpu
from jax.experimental.pallas import tpu_sc as plsc

NC = 2
NS = 16
K = 128


def _sc_mesh():
    return plsc.VectorSubcoreMesh(core_axis_name="c", subcore_axis_name="s")


def _make_deg_kernel(npad, epad, nbatch, d):
    rpt = npad // NS
    ept = epad // NS

    @functools.partial(
        pl.kernel,
        out_type=jax.ShapeDtypeStruct((NC, npad, d), jnp.float32),
        mesh=_sc_mesh(),
        scratch_types=[
            pltpu.VMEM_SHARED((npad, d), jnp.float32),
            pltpu.VMEM((nbatch, K), jnp.int32),
            pltpu.VMEM((K, d), jnp.float32),
            pltpu.SemaphoreType.DMA,
        ],
    )
    def deg_kernel(dst_hbm, zeros_hbm, ones_hbm, out_hbm,
                   acc, dstb, ones_v, sem):
        c = lax.axis_index("c")
        s = lax.axis_index("s")

        pltpu.sync_copy(zeros_hbm.at[pl.ds(s * rpt, rpt)],
                        acc.at[pl.ds(s * rpt, rpt)])
        pltpu.sync_copy(ones_hbm, ones_v)
        pltpu.sync_copy(dst_hbm.at[c, s], dstb)
        plsc.subcore_barrier()

        grp = 8

        @pl.loop(0, nbatch, step=grp)
        def _(i):
            for b in range(grp):
                @pl.when(i + b < nbatch)
                def _():
                    pltpu.async_copy(ones_v, acc.at[dstb.at[i + b]], sem,
                                     add=True)
            for b in range(grp):
                @pl.when(i + b < nbatch)
                def _():
                    pltpu.make_async_copy(ones_v, acc.at[dstb.at[i + b]],
                                          sem).wait()

        plsc.subcore_barrier()
        pltpu.sync_copy(acc.at[pl.ds(s * rpt, rpt)],
                        out_hbm.at[c].at[pl.ds(s * rpt, rpt)])

    return deg_kernel


def _make_edge_kernel(npad, epad, nbatch, d):
    rpt = npad // NS
    nrow = 2
    nidx = 4

    @functools.partial(
        pl.kernel,
        out_type=jax.ShapeDtypeStruct((NC * npad, d), jnp.float32),
        mesh=_sc_mesh(),
        scratch_types=[pltpu.VMEM_SHARED((npad, d), jnp.float32)]
        + [pltpu.VMEM((K,), jnp.int32)] * (2 * nidx)
        + [pltpu.VMEM((K, d), jnp.float32)] * nrow
        + [pltpu.SemaphoreType.DMA] * (nidx + 2 * nrow),
    )
    def edge_kernel(hp_hbm, src_hbm, dst_hbm, out_hbm, acc, *scr):
        srci = scr[:nidx]
        dsti = scr[nidx:2 * nidx]
        rows = scr[2 * nidx:2 * nidx + nrow]
        sem_i = scr[2 * nidx + nrow:2 * nidx + nrow + nidx]
        sem_g = scr[2 * nidx + nrow + nidx:2 * nidx + nrow + nidx + nrow]
        sem_s = scr[2 * nidx + nrow + nidx + nrow:]
        c = lax.axis_index("c")
        s = lax.axis_index("s")

        pltpu.sync_copy(hp_hbm.at[pl.ds(c * npad + s * rpt, rpt)],
                        acc.at[pl.ds(s * rpt, rpt)])

        srcc = src_hbm.at[c, s]
        dstc = dst_hbm.at[c, s]

        def load_idx(j, isl):
            pltpu.async_copy(srcc.at[j], srci[isl], sem_i[isl])
            pltpu.async_copy(dstc.at[j], dsti[isl], sem_i[isl])

        def wait_idx(j, isl):
            pltpu.make_async_copy(srcc.at[j], srci[isl], sem_i[isl]).wait()
            pltpu.make_async_copy(dstc.at[j], dsti[isl], sem_i[isl]).wait()

        def gather(isl, rb):
            pltpu.async_copy(hp_hbm.at[srci[isl]], rows[rb], sem_g[rb])

        def wait_gather(isl, rb):
            pltpu.make_async_copy(hp_hbm.at[srci[isl]], rows[rb],
                                  sem_g[rb]).wait()

        def scatter(isl, rb):
            pltpu.async_copy(rows[rb], acc.at[dsti[isl]], sem_s[rb],
                             add=True)

        def wait_scatter(isl, rb):
            pltpu.make_async_copy(rows[rb], acc.at[dsti[isl]],
                                  sem_s[rb]).wait()

        for j in range(min(3, nbatch)):
            load_idx(j, j)
        plsc.subcore_barrier()
        wait_idx(0, 0)
        gather(0, 0)

        @pl.loop(0, nbatch, step=nidx)
        def _(i):
            for b in range(nidx):
                j = i + b

                @pl.when(j < nbatch)
                def _():
                    @pl.when(j + 1 < nbatch)
                    def _():
                        wait_idx(j + 1, (b + 1) % nidx)

                        @pl.when(j >= 1)
                        def _():
                            wait_scatter((b + 3) % nidx, (b + 1) % nrow)

                        gather((b + 1) % nidx, (b + 1) % nrow)

                    @pl.when(j + 3 < nbatch)
                    def _():
                        load_idx(j + 3, (b + 3) % nidx)

                    wait_gather(b, b % nrow)
                    scatter(b, b % nrow)

        for j in (nbatch - 2, nbatch - 1):
            if j >= 0:
                wait_scatter(j % nidx, j % nrow)
        plsc.subcore_barrier()
        pltpu.sync_copy(acc.at[pl.ds(s * rpt, rpt)],
                        out_hbm.at[pl.ds(c * npad + s * rpt, rpt)])

    return edge_kernel


def _prologue_tc(x_blk, deg_blk, w0_blk, dinvb_ref, hp_ref, *, bn, n):
    i = pl.program_id(0)
    rows = lax.broadcasted_iota(jnp.int32, (bn, 1), 0) + i * bn
    valid = (rows < n).astype(jnp.float32)
    for b in range(2):
        deg = deg_blk[b, :, 0:1] + 1.0
        dinv = valid * lax.rsqrt(deg)
        dinvb = jnp.broadcast_to(dinv, (bn, x_blk.shape[2]))
        dinvb_ref[b] = dinvb
        h = jnp.dot(x_blk[b], w0_blk[b], preferred_element_type=jnp.float32)
        hp_ref[b] = h * dinvb


def _layer_tc(x_blk, acc_blk, dinvb_blk, b_blk, g_blk, be_blk, wn_blk,
              xn_ref, hp_ref):
    for b in range(2):
        c = acc_blk[b] * dinvb_blk[b] + b_blk[b][None, :]
        mu = jnp.mean(c, axis=-1, keepdims=True)
        var = jnp.mean((c - mu) ** 2, axis=-1, keepdims=True)
        cn = (c - mu) * lax.rsqrt(var + 1e-5) * g_blk[b][None, :] \
            + be_blk[b][None, :]
        xn = x_blk[b] + jnp.maximum(cn, 0.0)
        xn_ref[b] = xn
        hp_ref[b] = jnp.dot(xn, wn_blk[b],
                            preferred_element_type=jnp.float32) * dinvb_blk[b]


def _final_tc(x_blk, acc_blk, dinvb_blk, b_blk, g_blk, be_blk, wc_blk,
              bc_blk, out_ref):
    xs = []
    for b in range(2):
        c = acc_blk[b] * dinvb_blk[b] + b_blk[b][None, :]
        mu = jnp.mean(c, axis=-1, keepdims=True)
        var = jnp.mean((c - mu) ** 2, axis=-1, keepdims=True)
        cn = (c - mu) * lax.rsqrt(var + 1e-5) * g_blk[b][None, :] \
            + be_blk[b][None, :]
        xs.append(x_blk[b] + jnp.maximum(cn, 0.0))
    out = jnp.dot(xs[0], wc_blk[0], preferred_element_type=jnp.float32)
    out = out + jnp.dot(xs[1], wc_blk[1], preferred_element_type=jnp.float32)
    out_ref[...] = out + bc_blk[0][None, :]


def kernel(x_renormalized, edge_index_renormalized, x_vanilla,
           edge_index_vanilla, W_r, b_r, g_r, be_r, W_v, b_v, g_v, be_v,
           Wc, bc):
    n, d = x_renormalized.shape
    e = edge_index_renormalized.shape[1]
    nlayers = W_r.shape[0]
    nclass = Wc.shape[1]

    bn = 640
    npad = -(-n // bn) * bn
    nb = npad // bn
    epad = -(-e // (NS * K)) * (NS * K)
    nbatch = epad // (NS * K)

    f32 = jnp.float32

    def pad_edges(ei):
        src = jnp.pad(ei[0], (0, epad - e), constant_values=n)
        dst = jnp.pad(ei[1], (0, epad - e), constant_values=n)
        return src, dst

    src_r, dst_r = pad_edges(edge_index_renormalized)
    src_v, dst_v = pad_edges(edge_index_vanilla)
    src2 = jnp.stack([src_r, src_v + npad]).reshape(NC, NS, nbatch, K)
    dst2 = jnp.stack([dst_r, dst_v]).reshape(NC, NS, nbatch, K)

    x2 = jnp.stack([
        jnp.pad(x_renormalized, ((0, npad - n), (0, 0))),
        jnp.pad(x_vanilla, ((0, npad - n), (0, 0))),
    ])
    Ws = jnp.stack([W_r, W_v])
    bs = jnp.stack([b_r, b_v])
    gs = jnp.stack([g_r, g_v])
    bes = jnp.stack([be_r, be_v])
    Wc2 = jnp.stack([Wc[:d], Wc[d:]])
    bc2 = bc[None, :]

    deg_kernel = _make_deg_kernel(npad, epad, nbatch, d)
    edge_kernel = _make_edge_kernel(npad, epad, nbatch, d)

    deg16 = deg_kernel(dst2, jnp.zeros((npad, d), f32),
                       jnp.ones((K, d), f32))

    full2 = pl.BlockSpec((2, bn, d), lambda i: (0, i, 0))
    wfull = pl.BlockSpec((2, d, d), lambda i: (0, 0, 0))
    vec2 = pl.BlockSpec((2, d), lambda i: (0, 0))

    dinvb, hp = pl.pallas_call(
        functools.partial(_prologue_tc, bn=bn, n=n),
        grid=(nb,),
        in_specs=[full2, full2, wfull],
        out_specs=[full2, full2],
        out_shape=[jax.ShapeDtypeStruct((2, npad, d), f32),
                   jax.ShapeDtypeStruct((2, npad, d), f32)],
    )(x2, deg16, Ws[:, 0])

    for l in range(nlayers):
        acc = edge_kernel(hp.reshape(2 * npad, d), src2, dst2)
        acc = acc.reshape(2, npad, d)
        if l + 1 < nlayers:
            x2, hp = pl.pallas_call(
                _layer_tc,
                grid=(nb,),
                in_specs=[full2, full2, full2, vec2, vec2, vec2, wfull],
                out_specs=[full2, full2],
                out_shape=[jax.ShapeDtypeStruct((2, npad, d), f32),
                           jax.ShapeDtypeStruct((2, npad, d), f32)],
            )(x2, acc, dinvb, bs[:, l], gs[:, l], bes[:, l], Ws[:, l + 1])
        else:
            out = pl.pallas_call(
                _final_tc,
                grid=(nb,),
                in_specs=[full2, full2, full2, vec2, vec2, vec2,
                          pl.BlockSpec((2, d, nclass), lambda i: (0, 0, 0)),
                          pl.BlockSpec((1, nclass), lambda i: (0, 0))],
                out_specs=pl.BlockSpec((bn, nclass), lambda i: (i, 0)),
                out_shape=jax.ShapeDtypeStruct((npad, nclass), f32),
            )(x2, acc, dinvb, bs[:, l], gs[:, l], bes[:, l], Wc2, bc2)

    return out[:n]


# speedup vs baseline: 1.0117x; 1.0117x over previous
import functools

import jax
import jax.numpy as jnp
from jax import lax
from jax.experimental import pallas as pl
from jax.experimental.pallas import tpu as pltpu
from jax.experimental.pallas import tpu_sc as plsc

NC = 2
NS = 16
K = 128


def _sc_mesh():
    return plsc.VectorSubcoreMesh(core_axis_name="c", subcore_axis_name="s")


def _make_deg_kernel(npad, epad, nbatch, wd):
    rpt = npad // NS
    ept = epad // NS

    @functools.partial(
        pl.kernel,
        out_type=jax.ShapeDtypeStruct((NC, npad, wd), jnp.float32),
        mesh=_sc_mesh(),
        scratch_types=[
            pltpu.VMEM_SHARED((npad, wd), jnp.float32),
            pltpu.VMEM((nbatch, K), jnp.int32),
            pltpu.VMEM((K, wd), jnp.float32),
            pltpu.SemaphoreType.DMA,
        ],
    )
    def deg_kernel(dst_hbm, zeros_hbm, ones_hbm, out_hbm,
                   acc, dstb, ones_v, sem):
        c = lax.axis_index("c")
        s = lax.axis_index("s")

        pltpu.sync_copy(zeros_hbm.at[pl.ds(s * rpt, rpt)],
                        acc.at[pl.ds(s * rpt, rpt)])
        pltpu.sync_copy(ones_hbm, ones_v)
        pltpu.sync_copy(dst_hbm.at[c, s], dstb)
        plsc.subcore_barrier()

        grp = 8

        @pl.loop(0, nbatch, step=grp)
        def _(i):
            for b in range(grp):
                @pl.when(i + b < nbatch)
                def _():
                    pltpu.async_copy(ones_v, acc.at[dstb.at[i + b]], sem,
                                     add=True)
            for b in range(grp):
                @pl.when(i + b < nbatch)
                def _():
                    pltpu.make_async_copy(ones_v, acc.at[dstb.at[i + b]],
                                          sem).wait()

        plsc.subcore_barrier()
        pltpu.sync_copy(acc.at[pl.ds(s * rpt, rpt)],
                        out_hbm.at[c].at[pl.ds(s * rpt, rpt)])

    return deg_kernel


def _make_edge_kernel(npad, epad, nbatch, d):
    rpt = npad // NS
    nrow = 2
    nidx = 4

    @functools.partial(
        pl.kernel,
        out_type=jax.ShapeDtypeStruct((NC * npad, d), jnp.float32),
        mesh=_sc_mesh(),
        scratch_types=[pltpu.VMEM_SHARED((npad, d), jnp.float32)]
        + [pltpu.VMEM((K,), jnp.int32)] * (2 * nidx)
        + [pltpu.VMEM((K, d), jnp.float32)] * nrow
        + [pltpu.SemaphoreType.DMA] * (nidx + 2 * nrow),
    )
    def edge_kernel(hp_hbm, src_hbm, dst_hbm, out_hbm, acc, *scr):
        srci = scr[:nidx]
        dsti = scr[nidx:2 * nidx]
        rows = scr[2 * nidx:2 * nidx + nrow]
        sem_i = scr[2 * nidx + nrow:2 * nidx + nrow + nidx]
        sem_g = scr[2 * nidx + nrow + nidx:2 * nidx + nrow + nidx + nrow]
        sem_s = scr[2 * nidx + nrow + nidx + nrow:]
        c = lax.axis_index("c")
        s = lax.axis_index("s")

        pltpu.sync_copy(hp_hbm.at[pl.ds(c * npad + s * rpt, rpt)],
                        acc.at[pl.ds(s * rpt, rpt)])

        srcc = src_hbm.at[c, s]
        dstc = dst_hbm.at[c, s]

        def load_idx(j, isl):
            pltpu.async_copy(srcc.at[j], srci[isl], sem_i[isl])
            pltpu.async_copy(dstc.at[j], dsti[isl], sem_i[isl])

        def wait_idx(j, isl):
            pltpu.make_async_copy(srcc.at[j], srci[isl], sem_i[isl]).wait()
            pltpu.make_async_copy(dstc.at[j], dsti[isl], sem_i[isl]).wait()

        def gather(isl, rb):
            pltpu.async_copy(hp_hbm.at[srci[isl]], rows[rb], sem_g[rb])

        def wait_gather(isl, rb):
            pltpu.make_async_copy(hp_hbm.at[srci[isl]], rows[rb],
                                  sem_g[rb]).wait()

        def scatter(isl, rb):
            pltpu.async_copy(rows[rb], acc.at[dsti[isl]], sem_s[rb],
                             add=True)

        def wait_scatter(isl, rb):
            pltpu.make_async_copy(rows[rb], acc.at[dsti[isl]],
                                  sem_s[rb]).wait()

        for j in range(min(3, nbatch)):
            load_idx(j, j)
        plsc.subcore_barrier()
        wait_idx(0, 0)
        gather(0, 0)

        @pl.loop(0, nbatch, step=nidx)
        def _(i):
            for b in range(nidx):
                j = i + b

                @pl.when(j < nbatch)
                def _():
                    @pl.when(j + 1 < nbatch)
                    def _():
                        wait_idx(j + 1, (b + 1) % nidx)

                        @pl.when(j >= 1)
                        def _():
                            wait_scatter((b + 3) % nidx, (b + 1) % nrow)

                        gather((b + 1) % nidx, (b + 1) % nrow)

                    @pl.when(j + 3 < nbatch)
                    def _():
                        load_idx(j + 3, (b + 3) % nidx)

                    wait_gather(b, b % nrow)
                    scatter(b, b % nrow)

        for j in (nbatch - 2, nbatch - 1):
            if j >= 0:
                wait_scatter(j % nidx, j % nrow)
        plsc.subcore_barrier()
        pltpu.sync_copy(acc.at[pl.ds(s * rpt, rpt)],
                        out_hbm.at[pl.ds(c * npad + s * rpt, rpt)])

    return edge_kernel


def _h0_tc(x_blk, w0_blk, h0_ref):
    for b in range(2):
        h0_ref[b] = jnp.dot(x_blk[b], w0_blk[b],
                            preferred_element_type=jnp.float32)


def _scale_tc(h0_blk, deg_blk, dinvb_ref, hp_ref, *, bn, n):
    i = pl.program_id(0)
    rows = lax.broadcasted_iota(jnp.int32, (bn, 1), 0) + i * bn
    valid = (rows < n).astype(jnp.float32)
    for b in range(2):
        deg = deg_blk[b, :, 0:1] + 1.0
        dinv = valid * lax.rsqrt(deg)
        dinvb = jnp.broadcast_to(dinv, (bn, h0_blk.shape[2]))
        dinvb_ref[b] = dinvb
        hp_ref[b] = h0_blk[b] * dinvb


def _layer_tc(x_blk, acc_blk, dinvb_blk, b_blk, g_blk, be_blk, wn_blk,
              xn_ref, hp_ref):
    for b in range(2):
        c = acc_blk[b] * dinvb_blk[b] + b_blk[b][None, :]
        mu = jnp.mean(c, axis=-1, keepdims=True)
        var = jnp.mean((c - mu) ** 2, axis=-1, keepdims=True)
        cn = (c - mu) * lax.rsqrt(var + 1e-5) * g_blk[b][None, :] \
            + be_blk[b][None, :]
        xn = x_blk[b] + jnp.maximum(cn, 0.0)
        xn_ref[b] = xn
        hp_ref[b] = jnp.dot(xn, wn_blk[b],
                            preferred_element_type=jnp.float32) * dinvb_blk[b]


def _final_tc(x_blk, acc_blk, dinvb_blk, b_blk, g_blk, be_blk, wc_blk,
              bc_blk, out_ref):
    xs = []
    for b in range(2):
        c = acc_blk[b] * dinvb_blk[b] + b_blk[b][None, :]
        mu = jnp.mean(c, axis=-1, keepdims=True)
        var = jnp.mean((c - mu) ** 2, axis=-1, keepdims=True)
        cn = (c - mu) * lax.rsqrt(var + 1e-5) * g_blk[b][None, :] \
            + be_blk[b][None, :]
        xs.append(x_blk[b] + jnp.maximum(cn, 0.0))
    out = jnp.dot(xs[0], wc_blk[0], preferred_element_type=jnp.float32)
    out = out + jnp.dot(xs[1], wc_blk[1], preferred_element_type=jnp.float32)
    out_ref[...] = out + bc_blk[0][None, :]


def kernel(x_renormalized, edge_index_renormalized, x_vanilla,
           edge_index_vanilla, W_r, b_r, g_r, be_r, W_v, b_v, g_v, be_v,
           Wc, bc):
    n, d = x_renormalized.shape
    e = edge_index_renormalized.shape[1]
    nlayers = W_r.shape[0]
    nclass = Wc.shape[1]

    bn = 640
    npad = -(-n // bn) * bn
    nb = npad // bn
    epad = -(-e // (NS * K)) * (NS * K)
    nbatch = epad // (NS * K)

    f32 = jnp.float32

    def pad_edges(ei):
        src = jnp.pad(ei[0], (0, epad - e), constant_values=n)
        dst = jnp.pad(ei[1], (0, epad - e), constant_values=n)
        return src, dst

    src_r, dst_r = pad_edges(edge_index_renormalized)
    src_v, dst_v = pad_edges(edge_index_vanilla)
    src2 = jnp.stack([src_r, src_v + npad]).reshape(NC, NS, nbatch, K)
    dst2 = jnp.stack([dst_r, dst_v]).reshape(NC, NS, nbatch, K)

    x2 = jnp.stack([
        jnp.pad(x_renormalized, ((0, npad - n), (0, 0))),
        jnp.pad(x_vanilla, ((0, npad - n), (0, 0))),
    ])
    Ws = jnp.stack([W_r, W_v])
    bs = jnp.stack([b_r, b_v])
    gs = jnp.stack([g_r, g_v])
    bes = jnp.stack([be_r, be_v])
    Wc2 = jnp.stack([Wc[:d], Wc[d:]])
    bc2 = bc[None, :]

    deg_kernel = _make_deg_kernel(npad, epad, nbatch, d)
    edge_kernel = _make_edge_kernel(npad, epad, nbatch, d)

    full2 = pl.BlockSpec((2, bn, d), lambda i: (0, i, 0))
    wfull = pl.BlockSpec((2, d, d), lambda i: (0, 0, 0))
    vec2 = pl.BlockSpec((2, d), lambda i: (0, 0))

    deg16 = deg_kernel(dst2, jnp.zeros((npad, d), f32),
                       jnp.ones((K, d), f32))

    h0 = pl.pallas_call(
        _h0_tc,
        grid=(nb,),
        in_specs=[full2, wfull],
        out_specs=full2,
        out_shape=jax.ShapeDtypeStruct((2, npad, d), f32),
    )(x2, Ws[:, 0])

    dinvb, hp = pl.pallas_call(
        functools.partial(_scale_tc, bn=bn, n=n),
        grid=(nb,),
        in_specs=[full2, full2],
        out_specs=[full2, full2],
        out_shape=[jax.ShapeDtypeStruct((2, npad, d), f32),
                   jax.ShapeDtypeStruct((2, npad, d), f32)],
    )(h0, deg16)

    for l in range(nlayers):
        acc = edge_kernel(hp.reshape(2 * npad, d), src2, dst2)
        acc = acc.reshape(2, npad, d)
        if l + 1 < nlayers:
            x2, hp = pl.pallas_call(
                _layer_tc,
                grid=(nb,),
                in_specs=[full2, full2, full2, vec2, vec2, vec2, wfull],
                out_specs=[full2, full2],
                out_shape=[jax.ShapeDtypeStruct((2, npad, d), f32),
                           jax.ShapeDtypeStruct((2, npad, d), f32)],
            )(x2, acc, dinvb, bs[:, l], gs[:, l], bes[:, l], Ws[:, l + 1])
        else:
            out = pl.pallas_call(
                _final_tc,
                grid=(nb,),
                in_specs=[full2, full2, full2, vec2, vec2, vec2,
                          pl.BlockSpec((2, d, nclass), lambda i: (0, 0, 0)),
                          pl.BlockSpec((1, nclass), lambda i: (0, 0))],
                out_specs=pl.BlockSpec((bn, nclass), lambda i: (i, 0)),
                out_shape=jax.ShapeDtypeStruct((npad, nclass), f32),
            )(x2, acc, dinvb, bs[:, l], gs[:, l], bes[:, l], Wc2, bc2)

    return out[:n]
